# baseline (speedup 1.0000x reference)
import functools

import jax
import jax.numpy as jnp
from jax import lax
from jax.experimental import pallas as pl
from jax.experimental.pallas import tpu as pltpu
from jax.experimental.pallas import tpu_sc as plsc


_NW = 32
_CHUNK = 125
_NCHUNK = 8


def _sc_gather(ids3, reps, C, S, D):
    ncls = -(-C // _NW)

    mesh = plsc.VectorSubcoreMesh(core_axis_name="c", subcore_axis_name="s")

    @functools.partial(
        pl.kernel,
        mesh=mesh,
        out_type=jax.ShapeDtypeStruct((C, S, D), jnp.float32),
        scratch_types=[
            pltpu.VMEM((_NCHUNK, _CHUNK), jnp.int32),
            pltpu.VMEM((S, D), jnp.float32),
            pltpu.SemaphoreType.DMA,
        ],
    )
    def k(ids_hbm, reps_hbm, out_hbm, idx_v, rows_v, sem):
        wid = lax.axis_index("s") * 2 + lax.axis_index("c")

        def cls_body(t, carry):
            c = wid + t * _NW

            @pl.when(c < C)
            def _():
                pltpu.sync_copy(ids_hbm.at[c], idx_v)
                copies = [
                    pltpu.async_copy(
                        reps_hbm.at[idx_v.at[j]],
                        rows_v.at[pl.ds(j * _CHUNK, _CHUNK)],
                        sem,
                    )
                    for j in range(_NCHUNK)
                ]
                for cp in copies:
                    cp.wait()
                pltpu.sync_copy(rows_v, out_hbm.at[c])

            return carry

        lax.fori_loop(0, ncls, cls_body, 0)

    return k(ids3, reps)


_KPAD = 128


_PC = 4


def _tc_body(S, K, g_ref, idsc_ref, out_ref):
    for p in range(_PC):
        _tc_cls(S, K, g_ref, idsc_ref, out_ref, p)


def _tc_cls(S, K, g_ref, idsc_ref, out_ref, p):
    G = g_ref[p]
    csum = jnp.sum(G, axis=0, keepdims=True)
    center = csum / jnp.float32(S)

    sim_row = lax.dot_general(
        center, G, (((1,), (1,)), ((), ())))
    sim_col = lax.transpose(sim_row, (1, 0))

    i_col = lax.broadcasted_iota(jnp.int32, (S, 1), 0)
    j_row = lax.broadcasted_iota(jnp.int32, (1, S), 1)
    jlt = j_row < i_col
    lt = sim_row < sim_col
    eq = sim_row == sim_col
    cnt = jnp.where(lt | (eq & jlt), 1.0, 0.0)
    rank = jnp.sum(cnt, axis=1, keepdims=True)

    r_row = lax.broadcasted_iota(
        jnp.int32, (1, _KPAD), 1).astype(jnp.float32)
    sel = rank == r_row
    ids_col = idsc_ref[p]
    picked = jnp.where(sel, jnp.broadcast_to(ids_col, (S, _KPAD)), 0.0)
    out_ref[p] = jnp.sum(picked, axis=0, keepdims=True).astype(jnp.int32)


def _tc_compute(G3, ids3c, C, S, K):
    body = functools.partial(_tc_body, S, K)
    return pl.pallas_call(
        body,
        grid=(C // _PC,),
        in_specs=[
            pl.BlockSpec((_PC, S, G3.shape[2]), lambda c: (c, 0, 0)),
            pl.BlockSpec((_PC, S, 1), lambda c: (c, 0, 0)),
        ],
        out_specs=pl.BlockSpec((_PC, 1, _KPAD), lambda c: (c, 0, 0)),
        out_shape=jax.ShapeDtypeStruct((C, 1, _KPAD), jnp.int32),
    )(G3, ids3c)


def kernel(ids_per_cls_train, budget, feats, reps, d):
    ids = ids_per_cls_train.astype(jnp.int32)
    C, S = ids.shape
    D = reps.shape[1]
    K = min(100, S)

    G3 = _sc_gather(ids.reshape(C, _NCHUNK, _CHUNK), reps, C, S, D)
    ids3c = ids.astype(jnp.float32).reshape(C, S, 1)
    out = _tc_compute(G3, ids3c, C, S, K)

    return out[:, 0, :K].reshape(-1).astype(ids_per_cls_train.dtype)


# speedup vs baseline: 1.0103x; 1.0103x over previous
import functools

import jax
import jax.numpy as jnp
from jax import lax
from jax.experimental import pallas as pl
from jax.experimental.pallas import tpu as pltpu
from jax.experimental.pallas import tpu_sc as plsc


_NW = 32
_CHUNK = 100
_NCHUNK = 10
_WBLK = 200


def _sc_gather(ids3, reps, C, S, D):
    ncls = -(-C // _NW)

    mesh = plsc.VectorSubcoreMesh(core_axis_name="c", subcore_axis_name="s")

    nblk = S // _WBLK
    cpb = _WBLK // _CHUNK

    @functools.partial(
        pl.kernel,
        mesh=mesh,
        out_type=jax.ShapeDtypeStruct((C, S, D), jnp.float32),
        scratch_types=[
            pltpu.VMEM((_NCHUNK, _CHUNK), jnp.int32),
            pltpu.VMEM((2, _WBLK, D), jnp.float32),
            pltpu.SemaphoreType.DMA,
            pltpu.SemaphoreType.DMA,
            pltpu.SemaphoreType.DMA,
        ],
    )
    def k(ids_hbm, reps_hbm, out_hbm, idx_v, rows_v, gsem, wsem0, wsem1):
        wid = lax.axis_index("s") * 2 + lax.axis_index("c")
        wsems = (wsem0, wsem1)

        def drain(p):
            pltpu.make_async_copy(
                out_hbm.at[0].at[pl.ds(0, _WBLK)], rows_v.at[p], wsems[p]
            ).wait()

        def cls_body(t, carry):
            c = wid + t * _NW

            @pl.when(c < C)
            def _():
                pltpu.sync_copy(ids_hbm.at[c], idx_v)
                for b in range(nblk):
                    par = b % 2
                    if b >= 2:
                        drain(par)
                    copies = [
                        pltpu.async_copy(
                            reps_hbm.at[idx_v.at[b * cpb + j]],
                            rows_v.at[par].at[pl.ds(j * _CHUNK, _CHUNK)],
                            gsem,
                        )
                        for j in range(cpb)
                    ]
                    for cp in copies:
                        cp.wait()
                    pltpu.async_copy(
                        rows_v.at[par],
                        out_hbm.at[c].at[pl.ds(b * _WBLK, _WBLK)],
                        wsems[par],
                    )
                drain((nblk - 2) % 2)
                drain((nblk - 1) % 2)

            return carry

        lax.fori_loop(0, ncls, cls_body, 0)

    return k(ids3, reps)


_KPAD = 128


_PC = 4


def _tc_body(S, K, g_ref, idsc_ref, out_ref):
    for p in range(_PC):
        _tc_cls(S, K, g_ref, idsc_ref, out_ref, p)


def _tc_cls(S, K, g_ref, idsc_ref, out_ref, p):
    G = g_ref[p]
    csum = jnp.sum(G, axis=0, keepdims=True)
    center = csum / jnp.float32(S)

    sim_row = lax.dot_general(
        center, G, (((1,), (1,)), ((), ())))
    sim_col = lax.transpose(sim_row, (1, 0))

    i_col = lax.broadcasted_iota(jnp.int32, (S, 1), 0)
    j_row = lax.broadcasted_iota(jnp.int32, (1, S), 1)
    jlt = j_row < i_col
    lt = sim_row < sim_col
    eq = sim_row == sim_col
    cnt = jnp.where(lt | (eq & jlt), 1.0, 0.0)
    rank = jnp.sum(cnt, axis=1, keepdims=True)

    r_row = lax.broadcasted_iota(
        jnp.int32, (1, _KPAD), 1).astype(jnp.float32)
    sel = rank == r_row
    ids_col = idsc_ref[p]
    picked = jnp.where(sel, jnp.broadcast_to(ids_col, (S, _KPAD)), 0.0)
    out_ref[p] = jnp.sum(picked, axis=0, keepdims=True).astype(jnp.int32)


def _tc_compute(G3, ids3c, C, S, K):
    body = functools.partial(_tc_body, S, K)
    return pl.pallas_call(
        body,
        grid=(C // _PC,),
        in_specs=[
            pl.BlockSpec((_PC, S, G3.shape[2]), lambda c: (c, 0, 0)),
            pl.BlockSpec((_PC, S, 1), lambda c: (c, 0, 0)),
        ],
        out_specs=pl.BlockSpec((_PC, 1, _KPAD), lambda c: (c, 0, 0)),
        out_shape=jax.ShapeDtypeStruct((C, 1, _KPAD), jnp.int32),
    )(G3, ids3c)


def kernel(ids_per_cls_train, budget, feats, reps, d):
    ids = ids_per_cls_train.astype(jnp.int32)
    C, S = ids.shape
    D = reps.shape[1]
    K = min(100, S)

    G3 = _sc_gather(ids.reshape(C, _NCHUNK, _CHUNK), reps, C, S, D)
    ids3c = ids.astype(jnp.float32).reshape(C, S, 1)
    out = _tc_compute(G3, ids3c, C, S, K)

    return out[:, 0, :K].reshape(-1).astype(ids_per_cls_train.dtype)
